# baseline (speedup 1.0000x reference)
import functools

import jax
import jax.numpy as jnp
from jax import lax
from jax.experimental import pallas as pl
from jax.experimental.pallas import tpu as pltpu
from jax.experimental.pallas import tpu_sc as plsc


_B, _H, _W = 128, 512, 512
_BS = 8
_NC, _NS = 2, 16
_NW = _NC * _NS
_PB = _B // _NW
_RC = 64
_NCHUNK = _H // _RC
_NB = 3


def _tc_body(in_ref, r_ref, s_ref):
    x = in_ref[...]
    rolled = jnp.roll(x, 1, axis=2)
    r_ref[...] = rolled
    e = rolled[:, :, 0:2]
    s_ref[...] = jnp.concatenate([e[:, 511:512, :], e, e[:, 0:1, :]], axis=1)


def _sc_body(r_hbm, s_hbm, out_hbm, b0, b1, b2, sbuf, rt, rb,
             rsem, wsem, esem_r, esem_w):
    bufs = (b0, b1, b2)
    wid = lax.axis_index("s") * _NC + lax.axis_index("c")

    for j in range(_PB):
        b = wid * _PB + j

        e_reads = [
            pltpu.async_copy(s_hbm.at[pl.ds(b, 1)], sbuf, esem_r),
            pltpu.async_copy(
                r_hbm.at[pl.ds(b, 1), pl.ds(511, 1), pl.ds(0, _W)],
                rt, esem_r),
            pltpu.async_copy(
                r_hbm.at[pl.ds(b, 1), pl.ds(0, 1), pl.ds(0, _W)],
                rb, esem_r),
        ]

        rd = [None] * _NCHUNK
        wr = [None] * _NCHUNK
        for k in range(_NB):
            rd[k] = pltpu.async_copy(
                r_hbm.at[pl.ds(b, 1), pl.ds(k * _RC, _RC), pl.ds(0, _W)],
                bufs[k], rsem.at[k % _NB])
        for k in range(_NCHUNK):
            rd[k].wait()
            wr[k] = pltpu.async_copy(
                bufs[k % _NB],
                out_hbm.at[pl.ds(b, 1), pl.ds(k * _RC + 1, _RC), pl.ds(0, _W)],
                wsem.at[k % _NB])
            nk = k + _NB
            if nk < _NCHUNK:
                wr[k].wait()
                rd[nk] = pltpu.async_copy(
                    r_hbm.at[pl.ds(b, 1), pl.ds(nk * _RC, _RC), pl.ds(0, _W)],
                    bufs[nk % _NB], rsem.at[nk % _NB])

        for c in e_reads:
            c.wait()
        e_writes = [
            pltpu.async_copy(
                sbuf, out_hbm.at[pl.ds(b, 1), pl.ds(0, 514), pl.ds(512, 2)],
                esem_w),
            pltpu.async_copy(
                rt, out_hbm.at[pl.ds(b, 1), pl.ds(0, 1), pl.ds(0, _W)],
                esem_w),
            pltpu.async_copy(
                rb, out_hbm.at[pl.ds(b, 1), pl.ds(513, 1), pl.ds(0, _W)],
                esem_w),
        ]
        for k in range(max(_NCHUNK - _NB, 0), _NCHUNK):
            wr[k].wait()
        for c in e_writes:
            c.wait()


def kernel(inputs):
    b, h, w = inputs.shape
    rolled, strip = pl.pallas_call(
        _tc_body,
        grid=(b // _BS,),
        in_specs=[pl.BlockSpec((_BS, h, w), lambda i: (i, 0, 0))],
        out_specs=[
            pl.BlockSpec((_BS, h, w), lambda i: (i, 0, 0)),
            pl.BlockSpec((_BS, h + 2, 2), lambda i: (i, 0, 0)),
        ],
        out_shape=[
            jax.ShapeDtypeStruct((b, h, w), inputs.dtype),
            jax.ShapeDtypeStruct((b, h + 2, 2), inputs.dtype),
        ],
    )(inputs)

    mesh = plsc.VectorSubcoreMesh(core_axis_name="c", subcore_axis_name="s")
    f = functools.partial(
        pl.kernel,
        out_type=jax.ShapeDtypeStruct((b, h + 2, w + 2), inputs.dtype),
        mesh=mesh,
        compiler_params=pltpu.CompilerParams(use_tc_tiling_on_sc=False),
        scratch_types=[
            pltpu.VMEM((1, _RC, _W), inputs.dtype),
            pltpu.VMEM((1, _RC, _W), inputs.dtype),
            pltpu.VMEM((1, _RC, _W), inputs.dtype),
            pltpu.VMEM((1, 514, 2), inputs.dtype),
            pltpu.VMEM((1, 1, _W), inputs.dtype),
            pltpu.VMEM((1, 1, _W), inputs.dtype),
            pltpu.SemaphoreType.DMA((_NB,)),
            pltpu.SemaphoreType.DMA((_NB,)),
            pltpu.SemaphoreType.DMA,
            pltpu.SemaphoreType.DMA,
        ],
    )(_sc_body)
    return f(rolled, strip)


# speedup vs baseline: 2.5634x; 2.5634x over previous
import functools

import jax
import jax.numpy as jnp
from jax import lax
from jax.experimental import pallas as pl
from jax.experimental.pallas import tpu as pltpu
from jax.experimental.pallas import tpu_sc as plsc


_B, _H, _W = 128, 512, 512
_BS = 8
_NC, _NS = 2, 16
_NW = _NC * _NS
_PB = _B // _NW
_RC = 32
_NCHUNK = _H // _RC
_NB = 3


def _tc_body(in_ref, r_ref, s_ref):
    x = in_ref[...]
    rolled = jnp.roll(x, (1, 1), axis=(1, 2))
    r_ref[...] = rolled
    e = rolled[:, :, 0:2]
    s_ref[...] = jnp.concatenate([e, e[:, 0:2, :]], axis=1)


def _sc_body(r_hbm, s_hbm, out_hbm, b0, b1, b2, sbuf, rb,
             rsem, wsem, esem_r, esem_w):
    bufs = (b0, b1, b2)
    wid = lax.axis_index("s") * _NC + lax.axis_index("c")

    for j in range(_PB):
        b = wid * _PB + j

        e_reads = [
            pltpu.async_copy(s_hbm.at[pl.ds(b, 1)], sbuf, esem_r),
            pltpu.async_copy(
                r_hbm.at[pl.ds(b, 1), pl.ds(0, 2), pl.ds(0, _W)],
                rb, esem_r),
        ]

        rd = [None] * _NCHUNK
        wr = [None] * _NCHUNK
        for k in range(_NB):
            rd[k] = pltpu.async_copy(
                r_hbm.at[pl.ds(b, 1), pl.ds(k * _RC, _RC), pl.ds(0, _W)],
                bufs[k], rsem.at[k % _NB])
        for k in range(_NCHUNK):
            rd[k].wait()
            wr[k] = pltpu.async_copy(
                bufs[k % _NB],
                out_hbm.at[pl.ds(b, 1), pl.ds(k * _RC, _RC), pl.ds(0, _W)],
                wsem.at[k % _NB])
            nk = k + _NB
            if nk < _NCHUNK:
                wr[k].wait()
                rd[nk] = pltpu.async_copy(
                    r_hbm.at[pl.ds(b, 1), pl.ds(nk * _RC, _RC), pl.ds(0, _W)],
                    bufs[nk % _NB], rsem.at[nk % _NB])

        for c in e_reads:
            c.wait()
        e_writes = [
            pltpu.async_copy(
                sbuf, out_hbm.at[pl.ds(b, 1), pl.ds(0, 514), pl.ds(512, 2)],
                esem_w),
            pltpu.async_copy(
                rb, out_hbm.at[pl.ds(b, 1), pl.ds(512, 2), pl.ds(0, _W)],
                esem_w),
        ]
        for k in range(max(_NCHUNK - _NB, 0), _NCHUNK):
            wr[k].wait()
        for c in e_writes:
            c.wait()


def kernel(inputs):
    b, h, w = inputs.shape
    rolled, strip = pl.pallas_call(
        _tc_body,
        grid=(b // _BS,),
        in_specs=[pl.BlockSpec((_BS, h, w), lambda i: (i, 0, 0))],
        out_specs=[
            pl.BlockSpec((_BS, h, w), lambda i: (i, 0, 0)),
            pl.BlockSpec((_BS, h + 2, 2), lambda i: (i, 0, 0)),
        ],
        out_shape=[
            jax.ShapeDtypeStruct((b, h, w), inputs.dtype),
            jax.ShapeDtypeStruct((b, h + 2, 2), inputs.dtype),
        ],
    )(inputs)

    mesh = plsc.VectorSubcoreMesh(core_axis_name="c", subcore_axis_name="s")
    f = functools.partial(
        pl.kernel,
        out_type=jax.ShapeDtypeStruct((b, h + 2, w + 2), inputs.dtype),
        mesh=mesh,
        compiler_params=pltpu.CompilerParams(use_tc_tiling_on_sc=True),
        scratch_types=[
            pltpu.VMEM((1, _RC, _W), inputs.dtype),
            pltpu.VMEM((1, _RC, _W), inputs.dtype),
            pltpu.VMEM((1, _RC, _W), inputs.dtype),
            pltpu.VMEM((1, 514, 2), inputs.dtype),
            pltpu.VMEM((1, 2, _W), inputs.dtype),
            pltpu.SemaphoreType.DMA((_NB,)),
            pltpu.SemaphoreType.DMA((_NB,)),
            pltpu.SemaphoreType.DMA,
            pltpu.SemaphoreType.DMA,
        ],
    )(_sc_body)
    return f(rolled, strip)


# speedup vs baseline: 3.9987x; 1.5599x over previous
import jax
import jax.numpy as jnp
from jax.experimental import pallas as pl


_BS = 8


def _pad_body(in_ref, out_ref):
    x = in_ref[...]
    xr = jnp.concatenate([x[:, -1:, :], x, x[:, :1, :]], axis=1)
    out_ref[...] = jnp.concatenate([xr[:, :, -1:], xr, xr[:, :, :1]], axis=2)


def kernel(inputs):
    b, h, w = inputs.shape
    return pl.pallas_call(
        _pad_body,
        grid=(b // _BS,),
        in_specs=[pl.BlockSpec((_BS, h, w), lambda i: (i, 0, 0))],
        out_specs=pl.BlockSpec((_BS, h + 2, w + 2), lambda i: (i, 0, 0)),
        out_shape=jax.ShapeDtypeStruct((b, h + 2, w + 2), inputs.dtype),
    )(inputs)
